# baseline (speedup 1.0000x reference)
import jax
import jax.numpy as jnp
from jax.experimental import pallas as pl
from jax.experimental.pallas import tpu as pltpu

_L2 = 1.0
_LINF = 0.02
_K = 2048


def _body(y_ref, yh_ref, mask_ref, out_ref, sq_ref, hi_ref, lo_ref):
    B, N = y_ref.shape
    m = mask_ref[...]
    d = y_ref[...] * m - yh_ref[...] * m
    sq = d * d
    sq_ref[...] = sq
    total = jnp.sum(sq)

    bits = jax.lax.bitcast_convert_type(sq, jnp.int32)
    hi_ref[...] = (bits >> 16).astype(jnp.int16)
    lo_ref[...] = ((bits & 0xFFFF) - 0x8000).astype(jnp.int16)

    hi16 = hi_ref[...]
    lo16 = lo_ref[...]

    lo_a = jnp.zeros((B, 1), jnp.int32)
    hi_a = jnp.full((B, 1), jnp.int32(0x7F80))

    def step_a(_, carry):
        lo_b, hi_b = carry
        mid = lo_b + ((hi_b - lo_b + 1) >> 1)
        cnt = jnp.sum((hi16 >= mid.astype(jnp.int16)).astype(jnp.int32),
                      axis=1, keepdims=True)
        ge = cnt >= _K
        return jnp.where(ge, mid, lo_b), jnp.where(ge, hi_b, mid - 1)

    lo_a, hi_a = jax.lax.fori_loop(0, 15, step_a, (lo_a, hi_a))
    hstar = lo_a
    hstar16 = hstar.astype(jnp.int16)

    c_top = jnp.sum((hi16 > hstar16).astype(jnp.int32), axis=1, keepdims=True)
    kprime = _K - c_top
    in_plane = hi16 == hstar16

    lo_m = jnp.full((B, 1), jnp.int32(-0x8000))
    hi_m = jnp.full((B, 1), jnp.int32(0x7FFF))

    def step_b(_, carry):
        lo_b, hi_b = carry
        mid = lo_b + ((hi_b - lo_b + 1) >> 1)
        ok = in_plane & (lo16 >= mid.astype(jnp.int16))
        cnt = jnp.sum(ok.astype(jnp.int32), axis=1, keepdims=True)
        ge = cnt >= kprime
        return jnp.where(ge, mid, lo_b), jnp.where(ge, hi_b, mid - 1)

    lo_m, hi_m = jax.lax.fori_loop(0, 16, step_b, (lo_m, hi_m))
    mstar = lo_m

    t_bits = (hstar << 16) | ((mstar + 0x8000) & 0xFFFF)
    t = jax.lax.bitcast_convert_type(t_bits, jnp.float32)

    gt = in_plane & (lo16 > mstar.astype(jnp.int16)) | (hi16 > hstar16)
    s_gt = jnp.sum(jnp.where(gt, sq_ref[...], 0.0), axis=1, keepdims=True)
    c_gt = jnp.sum(gt.astype(jnp.int32), axis=1, keepdims=True)
    topk_sum = s_gt + (_K - c_gt).astype(jnp.float32) * t

    linf = jnp.sum(topk_sum) / B
    l2 = total / (B * N)
    out_ref[...] = jnp.reshape(_L2 * l2 + _LINF * linf, (1, 1))


@jax.jit
def kernel(y, yh, mask):
    B, N = y.shape
    res = pl.pallas_call(
        _body,
        out_shape=jax.ShapeDtypeStruct((1, 1), jnp.float32),
        scratch_shapes=[
            pltpu.VMEM((B, N), jnp.float32),
            pltpu.VMEM((B, N), jnp.int16),
            pltpu.VMEM((B, N), jnp.int16),
        ],
    )(y, yh, mask)
    return res[0, 0]


# speedup vs baseline: 1.5789x; 1.5789x over previous
import jax
import jax.numpy as jnp
from jax.experimental import pallas as pl

_L2 = 1.0
_LINF = 0.02
_K = 2048


def _body(y_ref, yh_ref, mask_ref, out_ref):
    B, N = y_ref.shape
    m = mask_ref[...]
    d = y_ref[...] * m - yh_ref[...] * m
    sq = d * d
    total = jnp.sum(sq)

    bits = jax.lax.bitcast_convert_type(sq, jnp.int32)

    lo = jnp.zeros((B, 1), jnp.int32)
    hi = jnp.full((B, 1), jnp.int32(0x7F800000))

    def step(_, carry):
        lo, hi = carry
        mid = lo + ((hi - lo + 1) >> 1)
        lt = jax.lax.shift_right_logical(bits - mid, 31)
        cnt = N - jnp.sum(lt, axis=1, keepdims=True)
        ge = cnt >= _K
        lo = jnp.where(ge, mid, lo)
        hi = jnp.where(ge, hi, mid - 1)
        return lo, hi

    lo, hi = jax.lax.fori_loop(0, 31, step, (lo, hi))
    t_bits = lo
    t = jax.lax.bitcast_convert_type(t_bits, jnp.float32)

    gt = bits > t_bits
    s_gt = jnp.sum(jnp.where(gt, sq, 0.0), axis=1, keepdims=True)
    c_gt = jnp.sum(gt.astype(jnp.int32), axis=1, keepdims=True)
    topk_sum = s_gt + (_K - c_gt).astype(jnp.float32) * t

    linf = jnp.sum(topk_sum) / B
    l2 = total / (B * N)
    out_ref[...] = jnp.reshape(_L2 * l2 + _LINF * linf, (1, 1))


@jax.jit
def kernel(y, yh, mask):
    res = pl.pallas_call(
        _body,
        out_shape=jax.ShapeDtypeStruct((1, 1), jnp.float32),
    )(y, yh, mask)
    return res[0, 0]


# speedup vs baseline: 1.6777x; 1.0625x over previous
import jax
import jax.numpy as jnp
from jax.experimental import pallas as pl

_L2 = 1.0
_LINF = 0.02
_K = 2048


def _body(y_ref, yh_ref, mask_ref, out_ref):
    B, N = y_ref.shape
    m = mask_ref[...]
    d = y_ref[...] * m - yh_ref[...] * m
    sq = d * d
    total = jnp.sum(sq)

    bits = jax.lax.bitcast_convert_type(sq, jnp.int32)

    lo = jnp.zeros((B, 1), jnp.int32)
    hi = jnp.full((B, 1), jnp.int32(0x7F800000))

    def step(_, carry):
        lo, hi = carry
        mid = lo + ((hi - lo + 1) >> 1)
        lt = jax.lax.shift_right_logical(bits - mid, 31)
        nsub = 8
        w = N // nsub
        parts = [jnp.sum(lt[:, i * w:(i + 1) * w], axis=1, keepdims=True)
                 for i in range(nsub)]
        while len(parts) > 1:
            parts = [parts[i] + parts[i + 1] for i in range(0, len(parts), 2)]
        cnt = N - parts[0]
        ge = cnt >= _K
        lo = jnp.where(ge, mid, lo)
        hi = jnp.where(ge, hi, mid - 1)
        return lo, hi

    lo, hi = jax.lax.fori_loop(0, 31, step, (lo, hi))
    t_bits = lo
    t = jax.lax.bitcast_convert_type(t_bits, jnp.float32)

    gt = bits > t_bits
    s_gt = jnp.sum(jnp.where(gt, sq, 0.0), axis=1, keepdims=True)
    c_gt = jnp.sum(gt.astype(jnp.int32), axis=1, keepdims=True)
    topk_sum = s_gt + (_K - c_gt).astype(jnp.float32) * t

    linf = jnp.sum(topk_sum) / B
    l2 = total / (B * N)
    out_ref[...] = jnp.reshape(_L2 * l2 + _LINF * linf, (1, 1))


@jax.jit
def kernel(y, yh, mask):
    res = pl.pallas_call(
        _body,
        out_shape=jax.ShapeDtypeStruct((1, 1), jnp.float32),
    )(y, yh, mask)
    return res[0, 0]
